# Initial kernel scaffold; baseline (speedup 1.0000x reference)
import jax
import jax.numpy as jnp
from jax.experimental import pallas as pl


def kernel(x, edge_index, edge_weight, W, b):
    raise NotImplementedError("write your pallas kernel here")


# speedup vs baseline: 2.3694x; 2.3694x over previous
import functools

import jax
import jax.numpy as jnp
from jax import lax
from jax.experimental import pallas as pl
from jax.experimental.pallas import tpu as pltpu
from jax.experimental.pallas import tpu_sc as plsc

N = 10000
E = 320000
D = 128
K = 16
ALPHA = 0.05
COEF = (1.0 - ALPHA) / K

NC = 2
NS = 16
NW = NC * NS

N2 = 10240
RPT = N2 // NS
NCHUNK = 81
EPT = NCHUNK * 128
E2 = NW * EPT

_GATHER_DNUMS = jax.lax.GatherDimensionNumbers(
    offset_dims=(), collapsed_slice_dims=(0,), start_index_map=(0,))


def _lane_bcast(v16, e):
    idx = jnp.full((16, 1), e, dtype=jnp.int32)
    return jax.lax.gather(v16, idx, _GATHER_DNUMS, (1,),
                          mode=jax.lax.GatherScatterMode.PROMISE_IN_BOUNDS)


def _step_body(cur_hbm, eidx_hbm, enrm_hbm, zeros_hbm,
               p0_hbm, p1_hbm,
               ebuf, nbuf, rowsv, agg, sem):
    cid = lax.axis_index("c")
    sid = lax.axis_index("s")
    wid = cid * NS + sid

    pltpu.sync_copy(zeros_hbm, agg.at[pl.ds(sid * RPT, RPT)])
    plsc.subcore_barrier()

    def chunk_body(j, _):
        pltpu.sync_copy(eidx_hbm.at[wid, j], ebuf)
        pltpu.sync_copy(enrm_hbm.at[wid, j], nbuf)
        pltpu.async_copy(cur_hbm.at[ebuf.at[0]], rowsv, sem).wait()

        def g_body(g, _):
            nv = nbuf[pl.ds(g * 16, 16)]
            for e in range(16):
                r = g * 16 + e
                s = _lane_bcast(nv, e)
                for f in range(8):
                    sl = pl.ds(f * 16, 16)
                    rowsv[r, sl] = rowsv[r, sl] * s
            return 0

        lax.fori_loop(0, 8, g_body, 0)

        pltpu.sync_copy(rowsv, agg.at[ebuf.at[1]], add=True)
        return 0

    lax.fori_loop(0, NCHUNK, chunk_body, 0)
    plsc.subcore_barrier()

    sl = pl.ds(sid * RPT, RPT)

    @pl.when(cid == 0)
    def _():
        pltpu.sync_copy(agg.at[sl], p0_hbm.at[sl])

    @pl.when(cid == 1)
    def _():
        pltpu.sync_copy(agg.at[sl], p1_hbm.at[sl])


_step = functools.partial(
    pl.kernel,
    out_type=(jax.ShapeDtypeStruct((N2, D), jnp.float32),
              jax.ShapeDtypeStruct((N2, D), jnp.float32)),
    mesh=plsc.VectorSubcoreMesh(core_axis_name="c", subcore_axis_name="s"),
    scratch_types=[
        pltpu.VMEM((2, 128), jnp.int32),
        pltpu.VMEM((128,), jnp.float32),
        pltpu.VMEM((128, D), jnp.float32),
        pltpu.VMEM_SHARED((N2, D), jnp.float32),
        pltpu.SemaphoreType.DMA,
    ],
)(_step_body)


def _combine_body(p0_ref, p1_ref, s_ref, cur_ref, so_ref):
    v = p0_ref[...] + p1_ref[...]
    cur_ref[...] = v
    so_ref[...] = s_ref[...] + v


def _combine(p0, p1, s):
    blk = 1024
    spec = pl.BlockSpec((blk, D), lambda i: (i, 0))
    return pl.pallas_call(
        _combine_body,
        grid=(N2 // blk,),
        in_specs=[spec, spec, spec],
        out_specs=(spec, spec),
        out_shape=(jax.ShapeDtypeStruct((N2, D), jnp.float32),
                   jax.ShapeDtypeStruct((N2, D), jnp.float32)),
    )(p0, p1, s)


def _final_body(x_ref, s_ref, wt_ref, b_ref, o_ref):
    h = ALPHA * x_ref[...] + COEF * s_ref[...]
    o_ref[...] = jnp.dot(h, wt_ref[...],
                         preferred_element_type=jnp.float32) + b_ref[...]


def _final(x, s, wt, b2):
    blk = 2000
    spec = pl.BlockSpec((blk, D), lambda i: (i, 0))
    return pl.pallas_call(
        _final_body,
        grid=(N // blk,),
        in_specs=[spec, spec,
                  pl.BlockSpec((D, D), lambda i: (0, 0)),
                  pl.BlockSpec((1, D), lambda i: (0, 0))],
        out_specs=spec,
        out_shape=jax.ShapeDtypeStruct((N, D), jnp.float32),
    )(x, s, wt, b2)


def kernel(x, edge_index, edge_weight, W, b):
    row, col = edge_index[0], edge_index[1]
    mask = row != col
    ew = jnp.where(mask, edge_weight, 0.0)
    loop_w = jnp.ones((N,), x.dtype).at[
        jnp.where(mask, N, row)].set(edge_weight, mode="drop")
    deg = jnp.zeros((N,), x.dtype).at[col].add(ew) + loop_w
    safe = deg > 0
    dinv = jnp.where(safe, lax.rsqrt(jnp.where(safe, deg, 1.0)), 0.0)
    norm_e = dinv[row] * ew * dinv[col]
    norm_loop = dinv * loop_w * dinv

    nodes = jnp.arange(N, dtype=jnp.int32)
    pad = E2 - (E + N)
    rows_all = jnp.concatenate(
        [row, nodes, jnp.zeros((pad,), jnp.int32)]).reshape(NW, NCHUNK, 128)
    cols_all = jnp.concatenate(
        [col, nodes, jnp.zeros((pad,), jnp.int32)]).reshape(NW, NCHUNK, 128)
    nrm_all = jnp.concatenate(
        [norm_e, norm_loop,
         jnp.zeros((pad,), jnp.float32)]).reshape(NW, NCHUNK, 128)
    eidx = jnp.stack([rows_all, cols_all], axis=2)

    cur0 = jnp.zeros((N2, D), jnp.float32).at[:N].set(x)
    s0 = jnp.zeros((N2, D), jnp.float32)
    zeros = jnp.zeros((RPT, D), jnp.float32)

    def k_body(_, carry):
        cur, s = carry
        p0, p1 = _step(cur, eidx, nrm_all, zeros)
        return _combine(p0, p1, s)

    _, s = lax.fori_loop(0, K, k_body, (cur0, s0))

    return _final(x, s[:N], W.T, b[None, :])
